# baseline (speedup 1.0000x reference)
import jax
import jax.numpy as jnp
from jax.experimental import pallas as pl
from jax.experimental.pallas import tpu as pltpu

T = 8192
D = 2048
E = 64
TOPK = 2
TILE_T = 512


def _router_kernel(x_ref, wg_ref, wu_ref, bias_ref, scale_ref,
                   w_out_ref, i_out_ref, wfull_ref):
    @pl.when(pl.program_id(0) == 0)
    def _init():
        wfull_ref[:, :E] = wg_ref[...]
        wfull_ref[:, E:] = wu_ref[...]

    acc = jax.lax.dot_general(
        wfull_ref[...], x_ref[...],
        dimension_numbers=(((0,), (1,)), ((), ())),
        preferred_element_type=jnp.float32,
    )
    gate = acc[:E, :]
    up = acc[E:, :]
    s = jnp.abs(up * gate * jax.nn.sigmoid(gate))
    ex = jnp.exp(jnp.minimum(s, 80.0))
    sm = ex / jnp.sum(ex, axis=0, keepdims=True)

    biased = sm + bias_ref[...]
    row = jax.lax.broadcasted_iota(jnp.int32, biased.shape, 0)

    m1 = jnp.max(biased, axis=0, keepdims=True)
    i1 = jnp.min(jnp.where(biased == m1, row, E), axis=0, keepdims=True)
    mask1 = row == i1
    rest = jnp.where(mask1, -jnp.inf, biased)
    m2 = jnp.max(rest, axis=0, keepdims=True)
    i2 = jnp.min(jnp.where(rest == m2, row, E), axis=0, keepdims=True)
    mask2 = row == i2

    w = 1.0 + sm * scale_ref[...]
    w1 = jnp.sum(jnp.where(mask1, w, 0.0), axis=0, keepdims=True)
    w2 = jnp.sum(jnp.where(mask2, w, 0.0), axis=0, keepdims=True)

    w_out_ref[...] = jnp.concatenate([w1, w2], axis=0).T
    i_out_ref[...] = jnp.concatenate([i1, i2], axis=0).T


@jax.jit
def kernel(x, W_gate, W_up, extra_scale, extra_bias):
    bias2d = extra_bias.reshape(E, 1)
    scale2d = extra_scale.reshape(E, 1)
    grid = (T // TILE_T,)
    weights, indices = pl.pallas_call(
        _router_kernel,
        grid=grid,
        in_specs=[
            pl.BlockSpec((TILE_T, D), lambda i: (i, 0)),
            pl.BlockSpec((D, E), lambda i: (0, 0)),
            pl.BlockSpec((D, E), lambda i: (0, 0)),
            pl.BlockSpec((E, 1), lambda i: (0, 0)),
            pl.BlockSpec((E, 1), lambda i: (0, 0)),
        ],
        out_specs=[
            pl.BlockSpec((TILE_T, TOPK), lambda i: (i, 0)),
            pl.BlockSpec((TILE_T, TOPK), lambda i: (i, 0)),
        ],
        out_shape=[
            jax.ShapeDtypeStruct((T, TOPK), jnp.float32),
            jax.ShapeDtypeStruct((T, TOPK), jnp.int32),
        ],
        scratch_shapes=[pltpu.VMEM((D, 2 * E), jnp.float32)],
    )(x, W_gate, W_up, bias2d, scale2d)
    return weights, indices


# speedup vs baseline: 1.2452x; 1.2452x over previous
import jax
import jax.numpy as jnp
from jax.experimental import pallas as pl
from jax.experimental.pallas import tpu as pltpu

T = 8192
D = 2048
E = 64
TOPK = 2
TILE_T = 512


def _router_kernel(x_ref, wg_ref, wu_ref, bias_ref, scale_ref,
                   w_out_ref, i_out_ref, wfull_ref):
    @pl.when(pl.program_id(0) == 0)
    def _init():
        wfull_ref[:, :E] = wg_ref[...]
        wfull_ref[:, E:] = wu_ref[...]

    acc = jax.lax.dot_general(
        wfull_ref[...], x_ref[...],
        dimension_numbers=(((0,), (1,)), ((), ())),
        preferred_element_type=jnp.float32,
    )
    gate = acc[:E, :]
    up = acc[E:, :]
    s = jnp.abs(up * gate * jax.nn.sigmoid(gate))
    ex = jnp.exp(jnp.minimum(s, 80.0))
    sm = ex / jnp.sum(ex, axis=0, keepdims=True)

    biased = sm + bias_ref[...]
    row = jax.lax.broadcasted_iota(jnp.int32, biased.shape, 0)

    m1 = jnp.max(biased, axis=0, keepdims=True)
    i1 = jnp.min(jnp.where(biased == m1, row, E), axis=0, keepdims=True)
    mask1 = row == i1
    rest = jnp.where(mask1, -jnp.inf, biased)
    m2 = jnp.max(rest, axis=0, keepdims=True)
    i2 = jnp.min(jnp.where(rest == m2, row, E), axis=0, keepdims=True)
    mask2 = row == i2

    w = 1.0 + sm * scale_ref[...]
    w1 = jnp.sum(jnp.where(mask1, w, 0.0), axis=0, keepdims=True)
    w2 = jnp.sum(jnp.where(mask2, w, 0.0), axis=0, keepdims=True)

    w_out_ref[...] = jnp.concatenate([w1, w2], axis=0)
    i_out_ref[...] = jnp.concatenate([i1, i2], axis=0)


@jax.jit
def kernel(x, W_gate, W_up, extra_scale, extra_bias):
    bias2d = extra_bias.reshape(E, 1)
    scale2d = extra_scale.reshape(E, 1)
    grid = (T // TILE_T,)
    weights, indices = pl.pallas_call(
        _router_kernel,
        grid=grid,
        in_specs=[
            pl.BlockSpec((TILE_T, D), lambda i: (i, 0)),
            pl.BlockSpec((D, E), lambda i: (0, 0)),
            pl.BlockSpec((D, E), lambda i: (0, 0)),
            pl.BlockSpec((E, 1), lambda i: (0, 0)),
            pl.BlockSpec((E, 1), lambda i: (0, 0)),
        ],
        out_specs=[
            pl.BlockSpec((TOPK, TILE_T), lambda i: (0, i)),
            pl.BlockSpec((TOPK, TILE_T), lambda i: (0, i)),
        ],
        out_shape=[
            jax.ShapeDtypeStruct((TOPK, T), jnp.float32),
            jax.ShapeDtypeStruct((TOPK, T), jnp.int32),
        ],
        scratch_shapes=[pltpu.VMEM((D, 2 * E), jnp.float32)],
    )(x, W_gate, W_up, bias2d, scale2d)
    return weights.T, indices.T


# speedup vs baseline: 1.3692x; 1.0996x over previous
import jax
import jax.numpy as jnp
from jax.experimental import pallas as pl
from jax.experimental.pallas import tpu as pltpu

T = 8192
D = 2048
E = 64
TOPK = 2
TILE_T = 1024


def _router_kernel(x_ref, wg_ref, wu_ref, bias_ref, scale_ref,
                   w_out_ref, i_out_ref, wfull_ref):
    @pl.when(pl.program_id(0) == 0)
    def _init():
        wfull_ref[:, :E] = wg_ref[...]
        wfull_ref[:, E:] = wu_ref[...]

    acc = jax.lax.dot_general(
        wfull_ref[...], x_ref[...],
        dimension_numbers=(((0,), (1,)), ((), ())),
        preferred_element_type=jnp.float32,
    )
    gate = acc[:E, :]
    up = acc[E:, :]
    s = jnp.abs(up * gate * jax.nn.sigmoid(gate))
    ex = jnp.exp(jnp.minimum(s, 80.0))
    sm = ex / jnp.sum(ex, axis=0, keepdims=True)

    biased = sm + bias_ref[...]
    row = jax.lax.broadcasted_iota(jnp.int32, biased.shape, 0)

    m1 = jnp.max(biased, axis=0, keepdims=True)
    i1 = jnp.min(jnp.where(biased == m1, row, E), axis=0, keepdims=True)
    mask1 = row == i1
    rest = jnp.where(mask1, -jnp.inf, biased)
    m2 = jnp.max(rest, axis=0, keepdims=True)
    i2 = jnp.min(jnp.where(rest == m2, row, E), axis=0, keepdims=True)
    mask2 = row == i2

    w = 1.0 + sm * scale_ref[...]
    w1 = jnp.sum(jnp.where(mask1, w, 0.0), axis=0, keepdims=True)
    w2 = jnp.sum(jnp.where(mask2, w, 0.0), axis=0, keepdims=True)

    w_out_ref[...] = jnp.concatenate([w1, w2], axis=0)
    i_out_ref[...] = jnp.concatenate([i1, i2], axis=0)


@jax.jit
def kernel(x, W_gate, W_up, extra_scale, extra_bias):
    bias2d = extra_bias.reshape(E, 1)
    scale2d = extra_scale.reshape(E, 1)
    grid = (T // TILE_T,)
    weights, indices = pl.pallas_call(
        _router_kernel,
        grid=grid,
        in_specs=[
            pl.BlockSpec((TILE_T, D), lambda i: (i, 0)),
            pl.BlockSpec((D, E), lambda i: (0, 0)),
            pl.BlockSpec((D, E), lambda i: (0, 0)),
            pl.BlockSpec((E, 1), lambda i: (0, 0)),
            pl.BlockSpec((E, 1), lambda i: (0, 0)),
        ],
        out_specs=[
            pl.BlockSpec((TOPK, TILE_T), lambda i: (0, i)),
            pl.BlockSpec((TOPK, TILE_T), lambda i: (0, i)),
        ],
        out_shape=[
            jax.ShapeDtypeStruct((TOPK, T), jnp.float32),
            jax.ShapeDtypeStruct((TOPK, T), jnp.int32),
        ],
        scratch_shapes=[pltpu.VMEM((D, 2 * E), jnp.float32)],
    )(x, W_gate, W_up, bias2d, scale2d)
    return weights.T, indices.T


# speedup vs baseline: 1.3705x; 1.0009x over previous
import jax
import jax.numpy as jnp
from jax.experimental import pallas as pl
from jax.experimental.pallas import tpu as pltpu

T = 8192
D = 2048
E = 64
TOPK = 2
TILE_T = 2048


def _router_kernel(x_ref, wg_ref, wu_ref, bias_ref, scale_ref,
                   w_out_ref, i_out_ref, wfull_ref):
    @pl.when(pl.program_id(0) == 0)
    def _init():
        wfull_ref[:, :E] = wg_ref[...]
        wfull_ref[:, E:] = wu_ref[...]

    acc = jax.lax.dot_general(
        wfull_ref[...], x_ref[...],
        dimension_numbers=(((0,), (1,)), ((), ())),
        preferred_element_type=jnp.float32,
    )
    gate = acc[:E, :]
    up = acc[E:, :]
    s = jnp.abs(up * gate * jax.nn.sigmoid(gate))
    ex = jnp.exp(jnp.minimum(s, 80.0))
    sm = ex / jnp.sum(ex, axis=0, keepdims=True)

    biased = sm + bias_ref[...]
    row = jax.lax.broadcasted_iota(jnp.int32, biased.shape, 0)

    m1 = jnp.max(biased, axis=0, keepdims=True)
    i1 = jnp.min(jnp.where(biased == m1, row, E), axis=0, keepdims=True)
    mask1 = row == i1
    rest = jnp.where(mask1, -jnp.inf, biased)
    m2 = jnp.max(rest, axis=0, keepdims=True)
    i2 = jnp.min(jnp.where(rest == m2, row, E), axis=0, keepdims=True)
    mask2 = row == i2

    w = 1.0 + sm * scale_ref[...]
    w1 = jnp.sum(jnp.where(mask1, w, 0.0), axis=0, keepdims=True)
    w2 = jnp.sum(jnp.where(mask2, w, 0.0), axis=0, keepdims=True)

    w_out_ref[...] = jnp.concatenate([w1, w2], axis=0)
    i_out_ref[...] = jnp.concatenate([i1, i2], axis=0)


@jax.jit
def kernel(x, W_gate, W_up, extra_scale, extra_bias):
    bias2d = extra_bias.reshape(E, 1)
    scale2d = extra_scale.reshape(E, 1)
    grid = (T // TILE_T,)
    weights, indices = pl.pallas_call(
        _router_kernel,
        grid=grid,
        in_specs=[
            pl.BlockSpec((TILE_T, D), lambda i: (i, 0)),
            pl.BlockSpec((D, E), lambda i: (0, 0)),
            pl.BlockSpec((D, E), lambda i: (0, 0)),
            pl.BlockSpec((E, 1), lambda i: (0, 0)),
            pl.BlockSpec((E, 1), lambda i: (0, 0)),
        ],
        out_specs=[
            pl.BlockSpec((TOPK, TILE_T), lambda i: (0, i)),
            pl.BlockSpec((TOPK, TILE_T), lambda i: (0, i)),
        ],
        out_shape=[
            jax.ShapeDtypeStruct((TOPK, T), jnp.float32),
            jax.ShapeDtypeStruct((TOPK, T), jnp.int32),
        ],
        scratch_shapes=[pltpu.VMEM((D, 2 * E), jnp.float32)],
    )(x, W_gate, W_up, bias2d, scale2d)
    return weights.T, indices.T
